# Initial kernel scaffold; baseline (speedup 1.0000x reference)
import jax
import jax.numpy as jnp
from jax.experimental import pallas as pl


def kernel(z, emb_weight):
    raise NotImplementedError("write your pallas kernel here")


# speedup vs baseline: 4.6752x; 4.6752x over previous
import functools

import jax
import jax.numpy as jnp
from jax.experimental import pallas as pl
from jax.experimental.pallas import tpu as pltpu

N_E = 8192
C_DIM = 32
BETA = 0.25
N_TOK = 4096

T = 512
K = 1024
NT = N_TOK // T
NK = N_E // K

_DOT_PREC = jax.lax.Precision.DEFAULT


def _argmin_body(z2_ref, e2_ref, z_ref, emb_ref, idx_ref, minv_s, mini_s):
    j = pl.program_id(1)

    @pl.when(j == 0)
    def _init():
        minv_s[...] = jnp.full((T, 1), jnp.inf, jnp.float32)
        mini_s[...] = jnp.zeros((T, 1), jnp.int32)

    m = jax.lax.dot_general(
        z_ref[...], emb_ref[...],
        dimension_numbers=(((1,), (1,)), ((), ())),
        preferred_element_type=jnp.float32,
        precision=_DOT_PREC,
    )
    d = (z2_ref[...] + e2_ref[...]) - 2.0 * m
    rowmin = jnp.min(d, axis=1, keepdims=True)
    col = jax.lax.broadcasted_iota(jnp.int32, (T, K), 1) + j * K
    rowidx = jnp.min(jnp.where(d == rowmin, col, N_E), axis=1, keepdims=True)
    upd = rowmin < minv_s[...]
    mini_s[...] = jnp.where(upd, rowidx, mini_s[...])
    minv_s[...] = jnp.where(upd, rowmin, minv_s[...])

    @pl.when(j == NK - 1)
    def _fin():
        idx_ref[...] = mini_s[...]


def _emit_body(idx_ref, z_ref, emb_ref,
               oh_ref, zq_ref, loss_ref, perp_ref,
               zq_s, hist_s, acc_s):
    i = pl.program_id(0)
    j = pl.program_id(1)

    idx = idx_ref[...]
    col = jax.lax.broadcasted_iota(jnp.int32, (T, K), 1) + j * K
    oh = (col == idx).astype(jnp.float32)
    oh_ref[...] = oh

    @pl.when(j == 0)
    def _init_zq():
        zq_s[...] = jnp.zeros((T, C_DIM), jnp.float32)

    zq_s[...] += jax.lax.dot_general(
        oh, emb_ref[...],
        dimension_numbers=(((1,), (0,)), ((), ())),
        preferred_element_type=jnp.float32,
        precision=_DOT_PREC,
    )

    colsum = jnp.sum(oh, axis=0, keepdims=True)

    @pl.when(i == 0)
    def _init_hist():
        hist_s[pl.ds(j, 1), :] = colsum

    @pl.when(i > 0)
    def _acc_hist():
        hist_s[pl.ds(j, 1), :] += colsum

    @pl.when(j == NK - 1)
    def _finish_row():
        z = z_ref[...]
        diff = zq_s[...] - z
        zq_ref[...] = z + diff
        sq = jnp.sum(diff * diff)

        @pl.when(i == 0)
        def _():
            acc_s[0] = sq

        @pl.when(i > 0)
        def _():
            acc_s[0] += sq

    @pl.when(jnp.logical_and(i == NT - 1, j == NK - 1))
    def _finalize():
        mse = acc_s[0] / jnp.float32(N_TOK * C_DIM)
        loss_ref[...] = jnp.full((1, 1), mse + BETA * mse, jnp.float32)
        e_mean = hist_s[...] / jnp.float32(N_TOK)
        ent = -jnp.sum(e_mean * jnp.log(e_mean + 1e-10))
        perp_ref[...] = jnp.full((1, 1), jnp.exp(ent), jnp.float32)


def _argmin_call(z2, e2, z_flat, emb_weight, interpret=False):
    return pl.pallas_call(
        _argmin_body,
        grid=(NT, NK),
        in_specs=[
            pl.BlockSpec((T, 1), lambda i, j: (i, 0)),
            pl.BlockSpec((1, K), lambda i, j: (0, j)),
            pl.BlockSpec((T, C_DIM), lambda i, j: (i, 0)),
            pl.BlockSpec((K, C_DIM), lambda i, j: (j, 0)),
        ],
        out_specs=pl.BlockSpec((T, 1), lambda i, j: (i, 0)),
        out_shape=jax.ShapeDtypeStruct((N_TOK, 1), jnp.int32),
        scratch_shapes=[
            pltpu.VMEM((T, 1), jnp.float32),
            pltpu.VMEM((T, 1), jnp.int32),
        ],
        interpret=interpret,
    )(z2, e2, z_flat, emb_weight)


def _emit_call(min_idx, z_flat, emb_weight, interpret=False):
    return pl.pallas_call(
        _emit_body,
        grid=(NT, NK),
        in_specs=[
            pl.BlockSpec((T, 1), lambda i, j: (i, 0)),
            pl.BlockSpec((T, C_DIM), lambda i, j: (i, 0)),
            pl.BlockSpec((K, C_DIM), lambda i, j: (j, 0)),
        ],
        out_specs=[
            pl.BlockSpec((T, K), lambda i, j: (i, j)),
            pl.BlockSpec((T, C_DIM), lambda i, j: (i, 0)),
            pl.BlockSpec((1, 1), lambda i, j: (0, 0)),
            pl.BlockSpec((1, 1), lambda i, j: (0, 0)),
        ],
        out_shape=[
            jax.ShapeDtypeStruct((N_TOK, N_E), jnp.float32),
            jax.ShapeDtypeStruct((N_TOK, C_DIM), jnp.float32),
            jax.ShapeDtypeStruct((1, 1), jnp.float32),
            jax.ShapeDtypeStruct((1, 1), jnp.float32),
        ],
        scratch_shapes=[
            pltpu.VMEM((T, C_DIM), jnp.float32),
            pltpu.VMEM((NK, K), jnp.float32),
            pltpu.SMEM((1,), jnp.float32),
        ],
        interpret=interpret,
    )(min_idx, z_flat, emb_weight)


def kernel(z, emb_weight, *, interpret=False):
    zp = jnp.transpose(z, (0, 2, 3, 1))
    z_flat = zp.reshape(-1, C_DIM)
    z2 = jnp.sum(z_flat ** 2, axis=1, keepdims=True)
    e2 = jnp.sum(emb_weight ** 2, axis=1)[None, :]

    min_idx = _argmin_call(z2, e2, z_flat, emb_weight, interpret=interpret)
    min_encodings, zq_st, loss, perp = _emit_call(
        min_idx, z_flat, emb_weight, interpret=interpret)

    z_q_out = jnp.transpose(zq_st.reshape(zp.shape), (0, 3, 1, 2))
    return (loss[0, 0], z_q_out, perp[0, 0], min_encodings, min_idx)


# speedup vs baseline: 6.6454x; 1.4214x over previous
import jax
import jax.numpy as jnp
from jax.experimental import pallas as pl
from jax.experimental.pallas import tpu as pltpu

N_E = 8192
C_DIM = 32
BETA = 0.25
N_TOK = 4096

T = 256
K = 1024
NT = N_TOK // T
NK = N_E // K

_DOT_PREC = jax.lax.Precision.DEFAULT


def _body(z2_ref, e2_ref, z_ref, emb2_ref, emb_ref,
          oh_ref, idx_ref, zq_ref, loss_ref, perp_ref,
          hist_s, acc_s):
    i = pl.program_id(0)
    z = z_ref[...]
    z2 = z2_ref[...]
    lcol = jax.lax.broadcasted_iota(jnp.int32, (T, K), 1)

    minv = jnp.full((T, 1), jnp.inf, jnp.float32)
    mini = jnp.zeros((T, 1), jnp.int32)
    for k in range(NK):
        m2 = jax.lax.dot_general(
            z, emb2_ref[k * K:(k + 1) * K, :],
            dimension_numbers=(((1,), (1,)), ((), ())),
            preferred_element_type=jnp.float32,
            precision=_DOT_PREC,
        )
        d = (z2 + e2_ref[:, k * K:(k + 1) * K]) - m2
        rowmin = jnp.min(d, axis=1, keepdims=True)
        rowidx = jnp.min(jnp.where(d == rowmin, lcol, K),
                         axis=1, keepdims=True) + k * K
        upd = rowmin < minv
        mini = jnp.where(upd, rowidx, mini)
        minv = jnp.where(upd, rowmin, minv)
    idx_ref[...] = mini

    zq = jnp.zeros((T, C_DIM), jnp.float32)
    colsums = []
    for k in range(NK):
        oh = (lcol == (mini - k * K)).astype(jnp.float32)
        oh_ref[:, k * K:(k + 1) * K] = oh
        zq = zq + jax.lax.dot_general(
            oh, emb_ref[k * K:(k + 1) * K, :],
            dimension_numbers=(((1,), (0,)), ((), ())),
            preferred_element_type=jnp.float32,
            precision=_DOT_PREC,
        )
        colsums.append(jnp.sum(oh, axis=0, keepdims=True))
    hrow = jnp.concatenate(colsums, axis=1)

    diff = zq - z
    zq_ref[...] = z + diff
    sq = jnp.sum(diff * diff)

    @pl.when(i == 0)
    def _init():
        hist_s[...] = hrow
        acc_s[0] = sq

    @pl.when(i > 0)
    def _acc():
        hist_s[...] += hrow
        acc_s[0] += sq

    @pl.when(i == NT - 1)
    def _finalize():
        mse = acc_s[0] / jnp.float32(N_TOK * C_DIM)
        loss_ref[...] = jnp.full((1, 1), mse + BETA * mse, jnp.float32)
        e_mean = hist_s[...] / jnp.float32(N_TOK)
        ent = -jnp.sum(e_mean * jnp.log(e_mean + 1e-10))
        perp_ref[...] = jnp.full((1, 1), jnp.exp(ent), jnp.float32)


def _vq_call(z2, e2, z_flat, emb2, emb_weight, interpret=False):
    return pl.pallas_call(
        _body,
        grid=(NT,),
        in_specs=[
            pl.BlockSpec((T, 1), lambda i: (i, 0)),
            pl.BlockSpec((1, N_E), lambda i: (0, 0)),
            pl.BlockSpec((T, C_DIM), lambda i: (i, 0)),
            pl.BlockSpec((N_E, C_DIM), lambda i: (0, 0)),
            pl.BlockSpec((N_E, C_DIM), lambda i: (0, 0)),
        ],
        out_specs=[
            pl.BlockSpec((T, N_E), lambda i: (i, 0)),
            pl.BlockSpec((T, 1), lambda i: (i, 0)),
            pl.BlockSpec((T, C_DIM), lambda i: (i, 0)),
            pl.BlockSpec((1, 1), lambda i: (0, 0)),
            pl.BlockSpec((1, 1), lambda i: (0, 0)),
        ],
        out_shape=[
            jax.ShapeDtypeStruct((N_TOK, N_E), jnp.float32),
            jax.ShapeDtypeStruct((N_TOK, 1), jnp.int32),
            jax.ShapeDtypeStruct((N_TOK, C_DIM), jnp.float32),
            jax.ShapeDtypeStruct((1, 1), jnp.float32),
            jax.ShapeDtypeStruct((1, 1), jnp.float32),
        ],
        scratch_shapes=[
            pltpu.VMEM((1, N_E), jnp.float32),
            pltpu.SMEM((1,), jnp.float32),
        ],
        interpret=interpret,
    )(z2, e2, z_flat, emb2, emb_weight)


def kernel(z, emb_weight, *, interpret=False):
    zp = jnp.transpose(z, (0, 2, 3, 1))
    z_flat = zp.reshape(-1, C_DIM)
    z2 = jnp.sum(z_flat ** 2, axis=1, keepdims=True)
    e2 = jnp.sum(emb_weight ** 2, axis=1)[None, :]
    emb2 = emb_weight * 2.0

    min_encodings, min_idx, zq_st, loss, perp = _vq_call(
        z2, e2, z_flat, emb2, emb_weight, interpret=interpret)

    z_q_out = jnp.transpose(zq_st.reshape(zp.shape), (0, 3, 1, 2))
    return (loss[0, 0], z_q_out, perp[0, 0], min_encodings, min_idx)


# speedup vs baseline: 6.6875x; 1.0063x over previous
import jax
import jax.numpy as jnp
from jax.experimental import pallas as pl
from jax.experimental.pallas import tpu as pltpu

N_E = 8192
C_DIM = 32
BETA = 0.25
N_TOK = 4096

T = 512
K = 1024
NT = N_TOK // T
NK = N_E // K

_DOT_PREC = jax.lax.Precision.DEFAULT


def _body(z2_ref, e2_ref, z_ref, emb2_ref, emb_ref,
          oh_ref, idx_ref, zq_ref, loss_ref, perp_ref,
          hist_s, acc_s):
    i = pl.program_id(0)
    z = z_ref[...]
    z2 = z2_ref[...]
    lcol = jax.lax.broadcasted_iota(jnp.int32, (T, K), 1)

    minv = jnp.full((T, 1), jnp.inf, jnp.float32)
    mini = jnp.zeros((T, 1), jnp.int32)
    for k in range(NK):
        m2 = jax.lax.dot_general(
            z, emb2_ref[k * K:(k + 1) * K, :],
            dimension_numbers=(((1,), (1,)), ((), ())),
            preferred_element_type=jnp.float32,
            precision=_DOT_PREC,
        )
        d = (z2 + e2_ref[:, k * K:(k + 1) * K]) - m2
        rowmin = jnp.min(d, axis=1, keepdims=True)
        rowidx = jnp.min(jnp.where(d == rowmin, lcol, K),
                         axis=1, keepdims=True) + k * K
        upd = rowmin < minv
        mini = jnp.where(upd, rowidx, mini)
        minv = jnp.where(upd, rowmin, minv)
    idx_ref[...] = mini

    zq = jnp.zeros((T, C_DIM), jnp.float32)
    colsums = []
    for k in range(NK):
        oh = (lcol == (mini - k * K)).astype(jnp.float32)
        oh_ref[:, k * K:(k + 1) * K] = oh
        zq = zq + jax.lax.dot_general(
            oh, emb_ref[k * K:(k + 1) * K, :],
            dimension_numbers=(((1,), (0,)), ((), ())),
            preferred_element_type=jnp.float32,
            precision=_DOT_PREC,
        )
        colsums.append(jnp.sum(oh, axis=0, keepdims=True))
    hrow = jnp.concatenate(colsums, axis=1)

    diff = zq - z
    zq_ref[...] = z + diff
    sq = jnp.sum(diff * diff)

    @pl.when(i == 0)
    def _init():
        hist_s[...] = hrow
        acc_s[0] = sq

    @pl.when(i > 0)
    def _acc():
        hist_s[...] += hrow
        acc_s[0] += sq

    @pl.when(i == NT - 1)
    def _finalize():
        mse = acc_s[0] / jnp.float32(N_TOK * C_DIM)
        loss_ref[...] = jnp.full((1, 1), mse + BETA * mse, jnp.float32)
        e_mean = hist_s[...] / jnp.float32(N_TOK)
        ent = -jnp.sum(e_mean * jnp.log(e_mean + 1e-10))
        perp_ref[...] = jnp.full((1, 1), jnp.exp(ent), jnp.float32)


def _vq_call(z2, e2, z_flat, emb2, emb_weight, interpret=False):
    return pl.pallas_call(
        _body,
        grid=(NT,),
        in_specs=[
            pl.BlockSpec((T, 1), lambda i: (i, 0)),
            pl.BlockSpec((1, N_E), lambda i: (0, 0)),
            pl.BlockSpec((T, C_DIM), lambda i: (i, 0)),
            pl.BlockSpec((N_E, C_DIM), lambda i: (0, 0)),
            pl.BlockSpec((N_E, C_DIM), lambda i: (0, 0)),
        ],
        out_specs=[
            pl.BlockSpec((T, N_E), lambda i: (i, 0)),
            pl.BlockSpec((T, 1), lambda i: (i, 0)),
            pl.BlockSpec((T, C_DIM), lambda i: (i, 0)),
            pl.BlockSpec((1, 1), lambda i: (0, 0)),
            pl.BlockSpec((1, 1), lambda i: (0, 0)),
        ],
        out_shape=[
            jax.ShapeDtypeStruct((N_TOK, N_E), jnp.float32),
            jax.ShapeDtypeStruct((N_TOK, 1), jnp.int32),
            jax.ShapeDtypeStruct((N_TOK, C_DIM), jnp.float32),
            jax.ShapeDtypeStruct((1, 1), jnp.float32),
            jax.ShapeDtypeStruct((1, 1), jnp.float32),
        ],
        scratch_shapes=[
            pltpu.VMEM((1, N_E), jnp.float32),
            pltpu.SMEM((1,), jnp.float32),
        ],
        interpret=interpret,
    )(z2, e2, z_flat, emb2, emb_weight)


def kernel(z, emb_weight, *, interpret=False):
    zp = jnp.transpose(z, (0, 2, 3, 1))
    z_flat = zp.reshape(-1, C_DIM)
    z2 = jnp.sum(z_flat ** 2, axis=1, keepdims=True)
    e2 = jnp.sum(emb_weight ** 2, axis=1)[None, :]
    emb2 = emb_weight * 2.0

    min_encodings, min_idx, zq_st, loss, perp = _vq_call(
        z2, e2, z_flat, emb2, emb_weight, interpret=interpret)

    z_q_out = jnp.transpose(zq_st.reshape(zp.shape), (0, 3, 1, 2))
    return (loss[0, 0], z_q_out, perp[0, 0], min_encodings, min_idx)


# speedup vs baseline: 7.2436x; 1.0832x over previous
import functools

import jax
import jax.numpy as jnp
from jax import lax
from jax.experimental import pallas as pl
from jax.experimental.pallas import tpu as pltpu
from jax.experimental.pallas import tpu_sc as plsc

N_E = 8192
C_DIM = 32
BETA = 0.25
N_TOK = 4096

T = 512
K = 1024
NT = N_TOK // T
NK = N_E // K

_DOT_PREC = jax.lax.Precision.DEFAULT


def _tc_body(z2_ref, e2_ref, z_ref, emb2_ref,
             oh_ref, idx_ref, loss_ref, acc_s):
    i = pl.program_id(0)
    z = z_ref[...]
    z2 = z2_ref[...]
    lcol = jax.lax.broadcasted_iota(jnp.int32, (T, K), 1)

    minv = jnp.full((T, 1), jnp.inf, jnp.float32)
    mini = jnp.zeros((T, 1), jnp.int32)
    for k in range(NK):
        m2 = jax.lax.dot_general(
            z, emb2_ref[k * K:(k + 1) * K, :],
            dimension_numbers=(((1,), (1,)), ((), ())),
            preferred_element_type=jnp.float32,
            precision=_DOT_PREC,
        )
        d = (z2 + e2_ref[:, k * K:(k + 1) * K]) - m2
        rowmin = jnp.min(d, axis=1, keepdims=True)
        rowidx = jnp.min(jnp.where(d == rowmin, lcol, K),
                         axis=1, keepdims=True) + k * K
        upd = rowmin < minv
        mini = jnp.where(upd, rowidx, mini)
        minv = jnp.where(upd, rowmin, minv)
    idx_ref[...] = mini

    for k in range(NK):
        oh_ref[:, k * K:(k + 1) * K] = (
            lcol == (mini - k * K)).astype(jnp.float32)

    sq = jnp.sum(minv)

    @pl.when(i == 0)
    def _init():
        acc_s[0] = sq

    @pl.when(i > 0)
    def _acc():
        acc_s[0] += sq

    @pl.when(i == NT - 1)
    def _finalize():
        mse = acc_s[0] / jnp.float32(N_TOK * C_DIM)
        loss_ref[...] = jnp.full((1, 1), mse + BETA * mse, jnp.float32)


def _tc_call(z2, e2, z_flat, emb2, interpret=False):
    return pl.pallas_call(
        _tc_body,
        grid=(NT,),
        in_specs=[
            pl.BlockSpec((T, 1), lambda i: (i, 0)),
            pl.BlockSpec((1, N_E), lambda i: (0, 0)),
            pl.BlockSpec((T, C_DIM), lambda i: (i, 0)),
            pl.BlockSpec((N_E, C_DIM), lambda i: (0, 0)),
        ],
        out_specs=[
            pl.BlockSpec((T, N_E), lambda i: (i, 0)),
            pl.BlockSpec((T, 1), lambda i: (i, 0)),
            pl.BlockSpec((1, 1), lambda i: (0, 0)),
        ],
        out_shape=[
            jax.ShapeDtypeStruct((N_TOK, N_E), jnp.float32),
            jax.ShapeDtypeStruct((N_TOK, 1), jnp.int32),
            jax.ShapeDtypeStruct((1, 1), jnp.float32),
        ],
        scratch_shapes=[
            pltpu.SMEM((1,), jnp.float32),
        ],
        interpret=interpret,
    )(z2, e2, z_flat, emb2)


_SC_INFO = plsc.get_sparse_core_info()
_NC = _SC_INFO.num_cores
_NS = _SC_INFO.num_subcores
_NW = _NC * _NS
_BPW = N_TOK // _NW
_HPW = N_E // _NS


def _sc_body(emb_hbm, idx_hbm, zq_hbm, hist_hbm,
             idx_v, rows_v, ones_v, chunk_v, hist_sh, sem):
    c = lax.axis_index("c")
    s = lax.axis_index("s")
    wid = s * _NC + c
    base = wid * _BPW

    pltpu.sync_copy(idx_hbm.at[pl.ds(base, _BPW)], idx_v)
    pltpu.async_copy(emb_hbm.at[idx_v], rows_v, sem).wait()
    pltpu.sync_copy(rows_v, zq_hbm.at[pl.ds(base, _BPW)])

    for t in range(_HPW // 16):
        chunk_v[pl.ds(t * 16, 16)] = jnp.zeros((16,), jnp.float32)
    for t in range(_BPW // 16):
        ones_v[pl.ds(t * 16, 16)] = jnp.ones((16,), jnp.float32)
    pltpu.sync_copy(chunk_v, hist_sh.at[pl.ds(s * _HPW, _HPW)])
    plsc.subcore_barrier()
    pltpu.sync_copy(ones_v, hist_sh.at[idx_v], add=True)
    plsc.subcore_barrier()
    pltpu.sync_copy(hist_sh.at[pl.ds(s * _HPW, _HPW)], chunk_v)
    pltpu.sync_copy(chunk_v, hist_hbm.at[c, pl.ds(s * _HPW, _HPW)])


@functools.partial(
    pl.kernel,
    mesh=plsc.VectorSubcoreMesh(core_axis_name="c", subcore_axis_name="s"),
    out_type=[
        jax.ShapeDtypeStruct((N_TOK, 128), jnp.float32),
        jax.ShapeDtypeStruct((_NC, N_E), jnp.float32),
    ],
    scratch_types=[
        pltpu.VMEM((_BPW,), jnp.int32),
        pltpu.VMEM((_BPW, 128), jnp.float32),
        pltpu.VMEM((_BPW,), jnp.float32),
        pltpu.VMEM((_HPW,), jnp.float32),
        pltpu.VMEM_SHARED((N_E,), jnp.float32),
        pltpu.SemaphoreType.DMA,
    ],
)
def _sc_call(emb_hbm, idx_hbm, zq_hbm, hist_hbm,
             idx_v, rows_v, ones_v, chunk_v, hist_sh, sem):
    _sc_body(emb_hbm, idx_hbm, zq_hbm, hist_hbm,
             idx_v, rows_v, ones_v, chunk_v, hist_sh, sem)


def _perp_body(hist_ref, perp_ref):
    h = hist_ref[...]
    e_mean = (h[0:1, :] + h[1:2, :]) / jnp.float32(N_TOK)
    ent = -jnp.sum(e_mean * jnp.log(e_mean + 1e-10))
    perp_ref[...] = jnp.full((1, 1), jnp.exp(ent), jnp.float32)


def _perp_call(hist, interpret=False):
    return pl.pallas_call(
        _perp_body,
        grid=(1,),
        in_specs=[pl.BlockSpec((_NC, N_E), lambda i: (0, 0))],
        out_specs=pl.BlockSpec((1, 1), lambda i: (0, 0)),
        out_shape=jax.ShapeDtypeStruct((1, 1), jnp.float32),
        interpret=interpret,
    )(hist)


def kernel(z, emb_weight, *, interpret=False):
    zp = jnp.transpose(z, (0, 2, 3, 1))
    z_flat = zp.reshape(-1, C_DIM)
    z2 = jnp.sum(z_flat ** 2, axis=1, keepdims=True)
    e2 = jnp.sum(emb_weight ** 2, axis=1)[None, :]
    emb2 = emb_weight * 2.0

    min_encodings, min_idx, loss = _tc_call(
        z2, e2, z_flat, emb2, interpret=interpret)

    emb_pad = jnp.pad(emb_weight, ((0, 0), (0, 128 - C_DIM)))
    zq_pad, hist = _sc_call(emb_pad, min_idx.reshape(-1))
    zq = zq_pad[:, :C_DIM]

    perp = _perp_call(hist, interpret=interpret)

    z_q_out = jnp.transpose(zq.reshape(zp.shape), (0, 3, 1, 2))
    return (loss[0, 0], z_q_out, perp[0, 0], min_encodings, min_idx)
